# baseline (speedup 1.0000x reference)
import numpy as np
import jax
import jax.numpy as jnp
from jax import lax
from jax.experimental import pallas as pl
from jax.experimental.pallas import tpu as pltpu
from jax.experimental.pallas import tpu_sc as plsc

_NUM_USERS = 100000
_POOL_SIZE = 200
_NUM_NEG = 50
_BATCH = 16384

_NC, _NS, _L = 2, 16, 16
_NW = _NC * _NS
_ROWS_PER_W = _BATCH // _NW
_EPW = _ROWS_PER_W * _NUM_NEG
_CHUNK = 128
_NCHUNK = _EPW // _CHUNK
_VPC = _CHUNK // _L

def _tf2x32(k0, k1, x0, x1):
    rot = ((13, 15, 26, 6), (17, 29, 16, 24))
    ks = (np.uint32(k0), np.uint32(k1),
          np.uint32(k0) ^ np.uint32(k1) ^ np.uint32(0x1BD11BDA))
    x0 = (x0 + ks[0]).astype(np.uint32)
    x1 = (x1 + ks[1]).astype(np.uint32)
    for i in range(5):
        for r in rot[i % 2]:
            x0 = (x0 + x1).astype(np.uint32)
            x1 = ((x1 << np.uint32(r)) | (x1 >> np.uint32(32 - r))).astype(np.uint32)
            x1 = x1 ^ x0
        x0 = (x0 + ks[(i + 1) % 3]).astype(np.uint32)
        x1 = (x1 + ks[(i + 2) % 3] + np.uint32(i + 1)).astype(np.uint32)
    return x0, x1


def _idx_k_host():
    def bits(k0, k1, size):
        hi = np.zeros(size, dtype=np.uint32)
        lo = np.arange(size, dtype=np.uint32)
        x0, x1 = _tf2x32(k0, k1, hi, lo)
        return x0 ^ x1

    a, b = _tf2x32(np.uint32(0), np.uint32(42),
                   np.zeros(2, dtype=np.uint32), np.arange(2, dtype=np.uint32))
    size = _BATCH * _NUM_NEG
    higher = bits(a[0], b[0], size)
    lower = bits(a[1], b[1], size)
    span = np.uint32(_POOL_SIZE)
    mult = np.uint32(((2 ** 16 % _POOL_SIZE) ** 2) % _POOL_SIZE)
    off = ((higher % span) * mult + lower % span) % span
    return off.astype(np.int32).reshape(_BATCH, _NUM_NEG)


_IDX_K = _idx_k_host()
_ROW_LOCAL = np.repeat((np.arange(_BATCH, dtype=np.int32) % _ROWS_PER_W),
                       _NUM_NEG)
_SEL_PACKED = (_ROW_LOCAL << 8) | _IDX_K.reshape(-1)

_MESH = plsc.VectorSubcoreMesh(
    core_axis_name="c", subcore_axis_name="s",
    num_cores=_NC, num_subcores=_NS)

_KERNEL_CFG = dict(
    out_type=jax.ShapeDtypeStruct((_BATCH * _NUM_NEG,), jnp.int32),
    mesh=_MESH,
    compiler_params=pltpu.CompilerParams(use_tc_tiling_on_sc=False,
                                         needs_layout_passes=False),
    scratch_types=[
        pltpu.VMEM((_ROWS_PER_W,), jnp.int32),
        pltpu.VMEM((_EPW,), jnp.int32),
        pltpu.VMEM((_EPW,), jnp.int32),
        pltpu.VMEM((_EPW,), jnp.int32),
        pltpu.SemaphoreType.DMA,
    ],
)


def _neg_gather_body(user_id_hbm, sel_hbm, pool_hbm, out_hbm,
                     uid_v, sel_v, fidx_v, out_v, sem):
    wid = lax.axis_index("s") * _NC + lax.axis_index("c")
    ebase = wid * _EPW
    pflat = pool_hbm

    pltpu.sync_copy(user_id_hbm.at[pl.ds(wid * _ROWS_PER_W, _ROWS_PER_W)],
                    uid_v)
    pltpu.sync_copy(sel_hbm.at[pl.ds(ebase, _EPW)], sel_v)

    def chunk(j, carry):
        base = j * _CHUNK
        for v in range(_VPC):
            sl = pl.ds(base + v * _L, _L)
            p = sel_v[sl]
            r = lax.shift_right_logical(p, 8)
            c = lax.bitwise_and(p, 255)
            u = plsc.load_gather(uid_v, [r])
            fidx_v[sl] = c * _NUM_USERS + u
        csl = pl.ds(base, _CHUNK)
        pltpu.async_copy(pflat.at[fidx_v.at[csl]], out_v.at[csl], sem)
        return carry

    lax.fori_loop(0, _NCHUNK, chunk, 0)
    pltpu.make_async_copy(pflat.at[pl.ds(0, _EPW)], out_v, sem).wait()
    pltpu.sync_copy(out_v, out_hbm.at[pl.ds(ebase, _EPW)])


_neg_gather = pl.kernel(_neg_gather_body, **_KERNEL_CFG)


def kernel(user_id, pool):
    pool_t_flat = pool.T.reshape(-1)
    neg_flat = _neg_gather(user_id, jnp.asarray(_SEL_PACKED), pool_t_flat)
    neg_items = neg_flat.reshape(_BATCH, _NUM_NEG)
    log_q = jnp.full((_BATCH, _NUM_NEG), -np.log(float(_POOL_SIZE)),
                     dtype=jnp.float32)
    return (neg_items, log_q)


# speedup vs baseline: 1.0940x; 1.0940x over previous
import numpy as np
import jax
import jax.numpy as jnp
from jax import lax
from jax.experimental import pallas as pl
from jax.experimental.pallas import tpu as pltpu
from jax.experimental.pallas import tpu_sc as plsc

_NUM_USERS = 100000
_POOL_SIZE = 200
_NUM_NEG = 50
_BATCH = 16384

_NC, _NS, _L = 2, 16, 16
_NW = _NC * _NS
_ROWS_PER_W = _BATCH // _NW
_EPW = _ROWS_PER_W * _NUM_NEG
_CHUNK = 128
_NCHUNK = _EPW // _CHUNK
_VPC = _CHUNK // _L

def _tf2x32(k0, k1, x0, x1):
    rot = ((13, 15, 26, 6), (17, 29, 16, 24))
    ks = (np.uint32(k0), np.uint32(k1),
          np.uint32(k0) ^ np.uint32(k1) ^ np.uint32(0x1BD11BDA))
    x0 = (x0 + ks[0]).astype(np.uint32)
    x1 = (x1 + ks[1]).astype(np.uint32)
    for i in range(5):
        for r in rot[i % 2]:
            x0 = (x0 + x1).astype(np.uint32)
            x1 = ((x1 << np.uint32(r)) | (x1 >> np.uint32(32 - r))).astype(np.uint32)
            x1 = x1 ^ x0
        x0 = (x0 + ks[(i + 1) % 3]).astype(np.uint32)
        x1 = (x1 + ks[(i + 2) % 3] + np.uint32(i + 1)).astype(np.uint32)
    return x0, x1


def _idx_k_host():
    def bits(k0, k1, size):
        hi = np.zeros(size, dtype=np.uint32)
        lo = np.arange(size, dtype=np.uint32)
        x0, x1 = _tf2x32(k0, k1, hi, lo)
        return x0 ^ x1

    a, b = _tf2x32(np.uint32(0), np.uint32(42),
                   np.zeros(2, dtype=np.uint32), np.arange(2, dtype=np.uint32))
    size = _BATCH * _NUM_NEG
    higher = bits(a[0], b[0], size)
    lower = bits(a[1], b[1], size)
    span = np.uint32(_POOL_SIZE)
    mult = np.uint32(((2 ** 16 % _POOL_SIZE) ** 2) % _POOL_SIZE)
    off = ((higher % span) * mult + lower % span) % span
    return off.astype(np.int32).reshape(_BATCH, _NUM_NEG)


_IDX_K = _idx_k_host()
_IDX_K3 = _IDX_K.reshape(_NW, _ROWS_PER_W, _NUM_NEG)
_RL3 = np.broadcast_to(
    np.arange(_ROWS_PER_W, dtype=np.int32)[None, None, :],
    (_NW, _NUM_NEG, _ROWS_PER_W))
_SEL_PACKED = ((_RL3 << 8)
               | np.transpose(_IDX_K3, (0, 2, 1))).reshape(-1)

_MESH = plsc.VectorSubcoreMesh(
    core_axis_name="c", subcore_axis_name="s",
    num_cores=_NC, num_subcores=_NS)

_KERNEL_CFG = dict(
    out_type=jax.ShapeDtypeStruct((_NUM_NEG, _BATCH), jnp.int32),
    mesh=_MESH,
    compiler_params=pltpu.CompilerParams(use_tc_tiling_on_sc=False,
                                         needs_layout_passes=False),
    scratch_types=[
        pltpu.VMEM((_ROWS_PER_W,), jnp.int32),
        pltpu.VMEM((_EPW,), jnp.int32),
        pltpu.VMEM((_EPW,), jnp.int32),
        pltpu.VMEM((_EPW,), jnp.int32),
        pltpu.SemaphoreType.DMA,
    ],
)


def _neg_gather_body(user_id_hbm, sel_hbm, pool_hbm, out_hbm,
                     uid_v, sel_v, fidx_v, out_v, sem):
    wid = lax.axis_index("s") * _NC + lax.axis_index("c")
    ebase = wid * _EPW
    pflat = pool_hbm

    pltpu.sync_copy(user_id_hbm.at[pl.ds(wid * _ROWS_PER_W, _ROWS_PER_W)],
                    uid_v)
    pltpu.sync_copy(sel_hbm.at[pl.ds(ebase, _EPW)], sel_v)

    def chunk(j, carry):
        base = j * _CHUNK
        for v in range(_VPC):
            sl = pl.ds(base + v * _L, _L)
            p = sel_v[sl]
            r = lax.shift_right_logical(p, 8)
            c = lax.bitwise_and(p, 255)
            u = plsc.load_gather(uid_v, [r])
            fidx_v[sl] = c * _NUM_USERS + u
        csl = pl.ds(base, _CHUNK)
        pltpu.async_copy(pflat.at[fidx_v.at[csl]], out_v.at[csl], sem)
        return carry

    lax.fori_loop(0, _NCHUNK, chunk, 0)
    pltpu.make_async_copy(pflat.at[pl.ds(0, _EPW)], out_v, sem).wait()

    def out_row(r, carry):
        pltpu.async_copy(out_v.at[pl.ds(r * _ROWS_PER_W, _ROWS_PER_W)],
                         out_hbm.at[r, pl.ds(wid * _ROWS_PER_W, _ROWS_PER_W)],
                         sem)
        return carry

    lax.fori_loop(0, _NUM_NEG, out_row, 0)
    pltpu.make_async_copy(pflat.at[pl.ds(0, _EPW)], out_v, sem).wait()


_neg_gather = pl.kernel(_neg_gather_body, **_KERNEL_CFG)


def kernel(user_id, pool):
    pool_t_flat = pool.T.reshape(-1)
    neg_t = _neg_gather(user_id, jnp.asarray(_SEL_PACKED), pool_t_flat)
    neg_items = neg_t.T
    log_q = jnp.full((_BATCH, _NUM_NEG), -np.log(float(_POOL_SIZE)),
                     dtype=jnp.float32)
    return (neg_items, log_q)
